# baseline (speedup 1.0000x reference)
import functools

import jax
import jax.numpy as jnp
from jax import lax
from jax.experimental import pallas as pl
from jax.experimental.pallas import tpu as pltpu
from jax.experimental.pallas import tpu_sc as plsc

T = 2048
D = 2048
E = 8
F = 1024
B = 256
NP = 4096 + E * B
NB = NP // B
FC = 512
NF = F // FC

NW = 32
TPW = T // NW
SUB = 16


def _router_body(x_ref, rw_ref, pos1_ref, pos2_ref, w1_ref, w2_ref,
                 be_ref, tot_ref):
    logits = lax.dot_general(
        x_ref[...], rw_ref[...], (((1,), (1,)), ((), ())),
        precision=lax.Precision.DEFAULT,
        preferred_element_type=jnp.float32)
    m = jnp.max(logits, axis=-1, keepdims=True)
    ex = jnp.exp(logits - m)
    probs = ex / jnp.sum(ex, axis=-1, keepdims=True)
    eiota = lax.broadcasted_iota(jnp.int32, (T, E), 1)
    m1 = jnp.max(probs, axis=-1, keepdims=True)
    i1 = jnp.min(jnp.where(probs == m1, eiota, E), axis=-1, keepdims=True)
    p2 = jnp.where(eiota == i1, -jnp.inf, probs)
    m2 = jnp.max(p2, axis=-1, keepdims=True)
    i2 = jnp.min(jnp.where(p2 == m2, eiota, E), axis=-1, keepdims=True)
    s = m1 + m2
    w1_ref[...] = m1 / s
    w2_ref[...] = m2 / s

    onehot1 = (eiota == i1).astype(jnp.float32)
    onehot2 = (eiota == i2).astype(jnp.float32)
    mask = onehot1 + onehot2
    r_iota = lax.broadcasted_iota(jnp.int32, (T, T), 0)
    c_iota = lax.broadcasted_iota(jnp.int32, (T, T), 1)
    ltri = (r_iota >= c_iota).astype(jnp.float32)
    incl = lax.dot_general(
        ltri, mask, (((1,), (0,)), ((), ())),
        precision=lax.Precision.DEFAULT,
        preferred_element_type=jnp.float32)

    counts = incl[T - 1:T, :].astype(jnp.int32)
    padded = ((counts + (B - 1)) // B) * B
    ei = lax.broadcasted_iota(jnp.int32, (E, E), 0)
    ej = lax.broadcasted_iota(jnp.int32, (E, E), 1)
    strict = (ei < ej).astype(jnp.float32)
    ex_off = lax.dot_general(
        padded.astype(jnp.float32), strict, (((1,), (0,)), ((), ())),
        precision=lax.Precision.DEFAULT,
        preferred_element_type=jnp.float32)
    tot = (ex_off[0:1, E - 1:E] + padded[0:1, E - 1:E].astype(jnp.float32))
    tot_ref[...] = jnp.broadcast_to(tot.astype(jnp.int32), (8, 1))

    posmat = ex_off + (incl - 1.0)
    pos1_ref[...] = jnp.sum(onehot1 * posmat, axis=1,
                            keepdims=True).astype(jnp.int32)
    pos2_ref[...] = jnp.sum(onehot2 * posmat, axis=1,
                            keepdims=True).astype(jnp.int32)

    biota = (lax.broadcasted_iota(jnp.int32, (NB, E), 0) * B).astype(
        jnp.float32)
    cmp = (ex_off <= biota).astype(jnp.int32)
    be_ref[...] = jnp.sum(cmp, axis=1, keepdims=True) - 1


def _gmm_body(be_sref, tot_sref, xs_ref, wg_ref, wv_ref, w2_ref, out_ref):
    i = pl.program_id(0)
    j = pl.program_id(1)

    @pl.when(j == 0)
    def _():
        out_ref[...] = jnp.zeros_like(out_ref)

    @pl.when(i * B < tot_sref[0])
    def _():
        xb = xs_ref[...]
        gate = lax.dot_general(
            xb, wg_ref[0], (((1,), (1,)), ((), ())),
            precision=lax.Precision.DEFAULT,
            preferred_element_type=jnp.float32)
        up = lax.dot_general(
            xb, wv_ref[0], (((1,), (1,)), ((), ())),
            precision=lax.Precision.DEFAULT,
            preferred_element_type=jnp.float32)
        act = gate * jax.nn.sigmoid(gate) * up
        out_ref[...] += lax.dot_general(
            act, w2_ref[0], (((1,), (1,)), ((), ())),
            precision=lax.Precision.DEFAULT,
            preferred_element_type=jnp.float32)


@functools.lru_cache(maxsize=None)
def _sc_kernels():
    mesh = plsc.VectorSubcoreMesh(core_axis_name="c", subcore_axis_name="s")

    @functools.partial(
        pl.kernel,
        mesh=mesh,
        compiler_params=pltpu.CompilerParams(needs_layout_passes=False),
        out_type=jax.ShapeDtypeStruct((NP, D), jnp.float32),
        scratch_types=[
            pltpu.VMEM((SUB,), jnp.int32),
            pltpu.VMEM((SUB,), jnp.int32),
            pltpu.VMEM((SUB, D), jnp.float32),
            pltpu.SemaphoreType.DMA,
        ],
    )
    def dispatch(x_hbm, pos1_hbm, pos2_hbm, xs_hbm, idx1_v, idx2_v, rows_v,
                 sem):
        wid = lax.axis_index("s") * 2 + lax.axis_index("c")
        base = wid * TPW
        for c in range(TPW // SUB):
            tb = base + c * SUB
            pltpu.sync_copy(pos1_hbm.at[pl.ds(tb, SUB)], idx1_v)
            pltpu.sync_copy(pos2_hbm.at[pl.ds(tb, SUB)], idx2_v)
            pltpu.sync_copy(x_hbm.at[pl.ds(tb, SUB)], rows_v)
            cp1 = pltpu.async_copy(rows_v, xs_hbm.at[idx1_v], sem)
            cp2 = pltpu.async_copy(rows_v, xs_hbm.at[idx2_v], sem)
            cp1.wait()
            cp2.wait()

    @functools.partial(
        pl.kernel,
        mesh=mesh,
        compiler_params=pltpu.CompilerParams(needs_layout_passes=False),
        out_type=jax.ShapeDtypeStruct((T, D), jnp.float32),
        scratch_types=[
            pltpu.VMEM((SUB,), jnp.int32),
            pltpu.VMEM((SUB,), jnp.int32),
            pltpu.VMEM((SUB,), jnp.float32),
            pltpu.VMEM((SUB,), jnp.float32),
            pltpu.VMEM((SUB, D), jnp.float32),
            pltpu.VMEM((SUB, D), jnp.float32),
            pltpu.SemaphoreType.DMA,
        ],
    )
    def combine(os_hbm, pos1_hbm, pos2_hbm, w1_hbm, w2_hbm, out_hbm,
                idx1_v, idx2_v, w1_v, w2_v, g1_v, g2_v, sem):
        wid = lax.axis_index("s") * 2 + lax.axis_index("c")
        base = wid * TPW
        for c in range(TPW // SUB):
            tb = base + c * SUB
            pltpu.sync_copy(pos1_hbm.at[pl.ds(tb, SUB)], idx1_v)
            pltpu.sync_copy(pos2_hbm.at[pl.ds(tb, SUB)], idx2_v)
            pltpu.sync_copy(w1_hbm.at[pl.ds(tb, SUB)], w1_v)
            pltpu.sync_copy(w2_hbm.at[pl.ds(tb, SUB)], w2_v)
            cp1 = pltpu.async_copy(os_hbm.at[idx1_v], g1_v, sem)
            cp2 = pltpu.async_copy(os_hbm.at[idx2_v], g2_v, sem)
            cp1.wait()
            cp2.wait()

            def row_body(r, carry):
                fr = jnp.full((SUB,), r, dtype=jnp.int32)
                w1r = plsc.load_gather(w1_v, [fr])
                w2r = plsc.load_gather(w2_v, [fr])

                def col_body(cc, carry2):
                    sl = pl.ds(cc * 16, 16)
                    g1_v[r, sl] = w1r * g1_v[r, sl] + w2r * g2_v[r, sl]
                    return carry2

                return lax.fori_loop(0, D // 16, col_body, carry)

            lax.fori_loop(0, SUB, row_body, 0)
            pltpu.sync_copy(g1_v, out_hbm.at[pl.ds(tb, SUB)])

    return dispatch, combine


@jax.jit
def kernel(hidden_states, router_weight, ws, w2s):
    x = hidden_states.reshape(T, D)
    pos1, pos2, w1, w2, be, tot = pl.pallas_call(
        _router_body,
        out_shape=[
            jax.ShapeDtypeStruct((T, 1), jnp.int32),
            jax.ShapeDtypeStruct((T, 1), jnp.int32),
            jax.ShapeDtypeStruct((T, 1), jnp.float32),
            jax.ShapeDtypeStruct((T, 1), jnp.float32),
            jax.ShapeDtypeStruct((NB, 1), jnp.int32),
            jax.ShapeDtypeStruct((8, 1), jnp.int32),
        ],
        in_specs=[pl.BlockSpec((T, D), lambda: (0, 0)),
                  pl.BlockSpec((E, D), lambda: (0, 0))],
        out_specs=[
            pl.BlockSpec((T, 1), lambda: (0, 0)),
            pl.BlockSpec((T, 1), lambda: (0, 0)),
            pl.BlockSpec((T, 1), lambda: (0, 0)),
            pl.BlockSpec((T, 1), lambda: (0, 0)),
            pl.BlockSpec((NB, 1), lambda: (0, 0)),
            pl.BlockSpec((8, 1), lambda: (0, 0)),
        ],
    )(x, router_weight)

    pos1 = pos1.reshape(T)
    pos2 = pos2.reshape(T)
    w1 = w1.reshape(T)
    w2 = w2.reshape(T)
    be = be.reshape(NB)
    tot = tot.reshape(8)[:1]

    dispatch, combine = _sc_kernels()
    xs = dispatch(x, pos1, pos2)

    grid_spec = pltpu.PrefetchScalarGridSpec(
        num_scalar_prefetch=2,
        grid=(NB, NF),
        in_specs=[
            pl.BlockSpec((B, D), lambda i, j, be_s, tot_s: (i, 0)),
            pl.BlockSpec((1, FC, D),
                         lambda i, j, be_s, tot_s: (be_s[i], j, 0)),
            pl.BlockSpec((1, FC, D),
                         lambda i, j, be_s, tot_s: (be_s[i], j + NF, 0)),
            pl.BlockSpec((1, D, FC),
                         lambda i, j, be_s, tot_s: (be_s[i], 0, j)),
        ],
        out_specs=pl.BlockSpec((B, D), lambda i, j, be_s, tot_s: (i, 0)),
    )
    os_ = pl.pallas_call(
        _gmm_body,
        grid_spec=grid_spec,
        out_shape=jax.ShapeDtypeStruct((NP, D), jnp.float32),
        compiler_params=pltpu.CompilerParams(
            dimension_semantics=("arbitrary", "arbitrary"),
        ),
    )(be, tot, xs, ws, ws, w2s)

    final = combine(os_, pos1, pos2, w1, w2)
    return final.reshape(hidden_states.shape)


# speedup vs baseline: 1.4531x; 1.4531x over previous
import functools

import jax
import jax.numpy as jnp
from jax import lax
from jax.experimental import pallas as pl
from jax.experimental.pallas import tpu as pltpu
from jax.experimental.pallas import tpu_sc as plsc

T = 2048
D = 2048
E = 8
F = 1024
B = 256
NP = 4096 + E * B
NB = NP // B

NW = 32
TPW = T // NW
SUB = 16


def _router_body(x_ref, rw_ref, pos1_ref, pos2_ref, w1_ref, w2_ref,
                 be_ref, tot_ref):
    logits = lax.dot_general(
        x_ref[...], rw_ref[...], (((1,), (1,)), ((), ())),
        precision=lax.Precision.DEFAULT,
        preferred_element_type=jnp.float32)
    m = jnp.max(logits, axis=-1, keepdims=True)
    ex = jnp.exp(logits - m)
    probs = ex / jnp.sum(ex, axis=-1, keepdims=True)
    eiota = lax.broadcasted_iota(jnp.int32, (T, E), 1)
    m1 = jnp.max(probs, axis=-1, keepdims=True)
    i1 = jnp.min(jnp.where(probs == m1, eiota, E), axis=-1, keepdims=True)
    p2 = jnp.where(eiota == i1, -jnp.inf, probs)
    m2 = jnp.max(p2, axis=-1, keepdims=True)
    i2 = jnp.min(jnp.where(p2 == m2, eiota, E), axis=-1, keepdims=True)
    s = m1 + m2
    w1_ref[...] = m1 / s
    w2_ref[...] = m2 / s

    onehot1 = (eiota == i1).astype(jnp.float32)
    onehot2 = (eiota == i2).astype(jnp.float32)
    mask = onehot1 + onehot2
    r_iota = lax.broadcasted_iota(jnp.int32, (T, T), 0)
    c_iota = lax.broadcasted_iota(jnp.int32, (T, T), 1)
    ltri = (r_iota >= c_iota).astype(jnp.float32)
    incl = lax.dot_general(
        ltri, mask, (((1,), (0,)), ((), ())),
        precision=lax.Precision.DEFAULT,
        preferred_element_type=jnp.float32)

    counts = incl[T - 1:T, :].astype(jnp.int32)
    padded = ((counts + (B - 1)) // B) * B
    ei = lax.broadcasted_iota(jnp.int32, (E, E), 0)
    ej = lax.broadcasted_iota(jnp.int32, (E, E), 1)
    strict = (ei < ej).astype(jnp.float32)
    ex_off = lax.dot_general(
        padded.astype(jnp.float32), strict, (((1,), (0,)), ((), ())),
        precision=lax.Precision.DEFAULT,
        preferred_element_type=jnp.float32)
    tot = (ex_off[0:1, E - 1:E] + padded[0:1, E - 1:E].astype(jnp.float32))
    tot_ref[...] = jnp.broadcast_to(tot.astype(jnp.int32), (8, 1))

    posmat = ex_off + (incl - 1.0)
    pos1_ref[...] = jnp.sum(onehot1 * posmat, axis=1,
                            keepdims=True).astype(jnp.int32)
    pos2_ref[...] = jnp.sum(onehot2 * posmat, axis=1,
                            keepdims=True).astype(jnp.int32)

    biota = (lax.broadcasted_iota(jnp.int32, (NB, E), 0) * B).astype(
        jnp.float32)
    cmp = (ex_off <= biota).astype(jnp.int32)
    be_ref[...] = jnp.sum(cmp, axis=1, keepdims=True) - 1


def _gmm_body(be_sref, tot_sref, xs_ref, wg_ref, wv_ref, w2_ref, out_ref):
    i = pl.program_id(0)

    @pl.when(i * B < tot_sref[0])
    def _():
        xb = xs_ref[...]
        gate = lax.dot_general(
            xb, wg_ref[0], (((1,), (1,)), ((), ())),
            precision=lax.Precision.DEFAULT,
            preferred_element_type=jnp.float32)
        up = lax.dot_general(
            xb, wv_ref[0], (((1,), (1,)), ((), ())),
            precision=lax.Precision.DEFAULT,
            preferred_element_type=jnp.float32)
        act = gate * jax.nn.sigmoid(gate) * up
        out_ref[...] = lax.dot_general(
            act, w2_ref[0], (((1,), (1,)), ((), ())),
            precision=lax.Precision.DEFAULT,
            preferred_element_type=jnp.float32)


@functools.lru_cache(maxsize=None)
def _sc_kernels():
    mesh = plsc.VectorSubcoreMesh(core_axis_name="c", subcore_axis_name="s")

    @functools.partial(
        pl.kernel,
        mesh=mesh,
        compiler_params=pltpu.CompilerParams(needs_layout_passes=False),
        out_type=jax.ShapeDtypeStruct((NP, D), jnp.float32),
        scratch_types=[
            pltpu.VMEM((SUB,), jnp.int32),
            pltpu.VMEM((SUB,), jnp.int32),
            pltpu.VMEM((SUB, D), jnp.float32),
            pltpu.SemaphoreType.DMA,
        ],
    )
    def dispatch(x_hbm, pos1_hbm, pos2_hbm, xs_hbm, idx1_v, idx2_v, rows_v,
                 sem):
        wid = lax.axis_index("s") * 2 + lax.axis_index("c")
        base = wid * TPW
        for c in range(TPW // SUB):
            tb = base + c * SUB
            pltpu.sync_copy(pos1_hbm.at[pl.ds(tb, SUB)], idx1_v)
            pltpu.sync_copy(pos2_hbm.at[pl.ds(tb, SUB)], idx2_v)
            pltpu.sync_copy(x_hbm.at[pl.ds(tb, SUB)], rows_v)
            cp1 = pltpu.async_copy(rows_v, xs_hbm.at[idx1_v], sem)
            cp2 = pltpu.async_copy(rows_v, xs_hbm.at[idx2_v], sem)
            cp1.wait()
            cp2.wait()

    @functools.partial(
        pl.kernel,
        mesh=mesh,
        compiler_params=pltpu.CompilerParams(needs_layout_passes=False),
        out_type=jax.ShapeDtypeStruct((T, D), jnp.float32),
        scratch_types=[
            pltpu.VMEM((SUB,), jnp.int32),
            pltpu.VMEM((SUB,), jnp.int32),
            pltpu.VMEM((SUB,), jnp.float32),
            pltpu.VMEM((SUB,), jnp.float32),
            pltpu.VMEM((SUB, D), jnp.float32),
            pltpu.VMEM((SUB, D), jnp.float32),
            pltpu.SemaphoreType.DMA,
        ],
    )
    def combine(os_hbm, pos1_hbm, pos2_hbm, w1_hbm, w2_hbm, out_hbm,
                idx1_v, idx2_v, w1_v, w2_v, g1_v, g2_v, sem):
        wid = lax.axis_index("s") * 2 + lax.axis_index("c")
        base = wid * TPW
        for c in range(TPW // SUB):
            tb = base + c * SUB
            pltpu.sync_copy(pos1_hbm.at[pl.ds(tb, SUB)], idx1_v)
            pltpu.sync_copy(pos2_hbm.at[pl.ds(tb, SUB)], idx2_v)
            pltpu.sync_copy(w1_hbm.at[pl.ds(tb, SUB)], w1_v)
            pltpu.sync_copy(w2_hbm.at[pl.ds(tb, SUB)], w2_v)
            cp1 = pltpu.async_copy(os_hbm.at[idx1_v], g1_v, sem)
            cp2 = pltpu.async_copy(os_hbm.at[idx2_v], g2_v, sem)
            cp1.wait()
            cp2.wait()

            def row_body(r, carry):
                fr = jnp.full((SUB,), r, dtype=jnp.int32)
                w1r = plsc.load_gather(w1_v, [fr])
                w2r = plsc.load_gather(w2_v, [fr])

                @plsc.parallel_loop(0, D // 16, unroll=8)
                def _(cc):
                    sl = pl.ds(cc * 16, 16)
                    g1_v[r, sl] = w1r * g1_v[r, sl] + w2r * g2_v[r, sl]

                return carry

            lax.fori_loop(0, SUB, row_body, 0)
            pltpu.sync_copy(g1_v, out_hbm.at[pl.ds(tb, SUB)])

    return dispatch, combine


@jax.jit
def kernel(hidden_states, router_weight, ws, w2s):
    x = hidden_states.reshape(T, D)
    pos1, pos2, w1, w2, be, tot = pl.pallas_call(
        _router_body,
        out_shape=[
            jax.ShapeDtypeStruct((T, 1), jnp.int32),
            jax.ShapeDtypeStruct((T, 1), jnp.int32),
            jax.ShapeDtypeStruct((T, 1), jnp.float32),
            jax.ShapeDtypeStruct((T, 1), jnp.float32),
            jax.ShapeDtypeStruct((NB, 1), jnp.int32),
            jax.ShapeDtypeStruct((8, 1), jnp.int32),
        ],
        in_specs=[pl.BlockSpec((T, D), lambda: (0, 0)),
                  pl.BlockSpec((E, D), lambda: (0, 0))],
        out_specs=[
            pl.BlockSpec((T, 1), lambda: (0, 0)),
            pl.BlockSpec((T, 1), lambda: (0, 0)),
            pl.BlockSpec((T, 1), lambda: (0, 0)),
            pl.BlockSpec((T, 1), lambda: (0, 0)),
            pl.BlockSpec((NB, 1), lambda: (0, 0)),
            pl.BlockSpec((8, 1), lambda: (0, 0)),
        ],
    )(x, router_weight)

    pos1 = pos1.reshape(T)
    pos2 = pos2.reshape(T)
    w1 = w1.reshape(T)
    w2 = w2.reshape(T)
    be = be.reshape(NB)
    tot = tot.reshape(8)[:1]

    dispatch, combine = _sc_kernels()
    xs = dispatch(x, pos1, pos2)

    grid_spec = pltpu.PrefetchScalarGridSpec(
        num_scalar_prefetch=2,
        grid=(NB,),
        in_specs=[
            pl.BlockSpec((B, D), lambda i, be_s, tot_s: (i, 0)),
            pl.BlockSpec((1, F, D), lambda i, be_s, tot_s: (be_s[i], 0, 0)),
            pl.BlockSpec((1, F, D), lambda i, be_s, tot_s: (be_s[i], 1, 0)),
            pl.BlockSpec((1, D, F), lambda i, be_s, tot_s: (be_s[i], 0, 0)),
        ],
        out_specs=pl.BlockSpec((B, D), lambda i, be_s, tot_s: (i, 0)),
    )
    os_ = pl.pallas_call(
        _gmm_body,
        grid_spec=grid_spec,
        out_shape=jax.ShapeDtypeStruct((NP, D), jnp.float32),
        compiler_params=pltpu.CompilerParams(
            dimension_semantics=("arbitrary",),
        ),
    )(be, tot, xs, ws, ws, w2s)

    final = combine(os_, pos1, pos2, w1, w2)
    return final.reshape(hidden_states.shape)
